# baseline (speedup 1.0000x reference)
import functools

import jax
import jax.numpy as jnp
from jax import lax
from jax.experimental import pallas as pl
from jax.experimental.pallas import tpu as pltpu
from jax.experimental.pallas import tpu_sc as plsc

MIN_POS = -1.2
MAX_POS = 0.6
MAX_SPEED = 0.07

N = 16384
NC = 2
NS = 16
NW = NC * NS
CHUNK = N // NW
LANES = 16
IDX_BLK = 128
NBLK = CHUNK // IDX_BLK
GRP_PER_BLK = IDX_BLK // LANES

_B0 = float(-MIN_POS)
_B1 = float(MAX_SPEED)
_M0 = float(1023.999 / (MAX_POS - MIN_POS))
_M1 = float(1023.999 / (2.0 * MAX_SPEED))

_mesh = plsc.VectorSubcoreMesh(core_axis_name="c", subcore_axis_name="s")


@functools.partial(
    pl.kernel,
    mesh=_mesh,
    out_type=jax.ShapeDtypeStruct((N,), jnp.float32),
    scratch_types=[
        pltpu.VMEM((4, CHUNK), jnp.float32),
        pltpu.SemaphoreType.DMA,
        pltpu.SemaphoreType.DMA,
        pltpu.SemaphoreType.DMA,
    ],
)
def _sc_lookup(inp_t_hbm, data_hbm, out_hbm, scr, sem_in, sem_g, sem_o):
    wid = lax.axis_index("s") * NC + lax.axis_index("c")
    base = wid * CHUNK
    scr_i = scr.bitcast(jnp.int32)

    in_cps = []
    for j in range(NBLK):
        blk = pl.ds(j * IDX_BLK, IDX_BLK)
        in_cps.append((
            pltpu.async_copy(
                inp_t_hbm.at[0, pl.ds(base + j * IDX_BLK, IDX_BLK)],
                scr.at[0, blk], sem_in),
            pltpu.async_copy(
                inp_t_hbm.at[1, pl.ds(base + j * IDX_BLK, IDX_BLK)],
                scr.at[1, blk], sem_in),
        ))

    b0 = jnp.float32(_B0)
    b1 = jnp.float32(_B1)
    m0 = jnp.float32(_M0)
    m1 = jnp.float32(_M1)

    flat = data_hbm.at[0, pl.ds(0, IDX_BLK)]

    g_cps = [None] * NBLK
    o_cps = [None] * NBLK
    for j in range(NBLK):
        blk = pl.ds(j * IDX_BLK, IDX_BLK)
        cp_p, cp_v = in_cps[j]
        cp_p.wait()
        cp_v.wait()
        for g in range(j * GRP_PER_BLK, (j + 1) * GRP_PER_BLK):
            grp = pl.ds(g * LANES, LANES)
            pos = scr[0, grp]
            vel = scr[1, grp]
            r = ((pos + b0) * m0).astype(jnp.int32)
            c = ((vel + b1) * m1).astype(jnp.int32)
            scr_i[2, grp] = (
                ((r >> 3) << 13) + ((r & 7) << 7) + ((c >> 7) << 10) + (c & 127)
            )
        g_cps[j] = pltpu.async_copy(
            flat.at[scr_i.at[2, blk]], scr.at[3, blk], sem_g)
        if j > 0:
            g_cps[j - 1].wait()
            o_cps[j - 1] = pltpu.async_copy(
                scr.at[3, pl.ds((j - 1) * IDX_BLK, IDX_BLK)],
                out_hbm.at[pl.ds(base + (j - 1) * IDX_BLK, IDX_BLK)], sem_o)

    g_cps[NBLK - 1].wait()
    o_cps[NBLK - 1] = pltpu.async_copy(
        scr.at[3, pl.ds((NBLK - 1) * IDX_BLK, IDX_BLK)],
        out_hbm.at[pl.ds(base + (NBLK - 1) * IDX_BLK, IDX_BLK)], sem_o)
    for cp in o_cps:
        cp.wait()


def kernel(inp, data):
    return _sc_lookup(inp.T, data)


# speedup vs baseline: 1.0373x; 1.0373x over previous
import functools

import jax
import jax.numpy as jnp
from jax import lax
from jax.experimental import pallas as pl
from jax.experimental.pallas import tpu as pltpu
from jax.experimental.pallas import tpu_sc as plsc

MIN_POS = -1.2
MAX_POS = 0.6
MAX_SPEED = 0.07

N = 16384
NC = 2
NS = 16
NW = NC * NS
CHUNK = N // NW
LANES = 16
IDX_BLK = 128
NBLK = CHUNK // IDX_BLK
GRP_PER_BLK = IDX_BLK // LANES

_B0 = float(-MIN_POS)
_B1 = float(MAX_SPEED)
_M0 = float(1023.999 / (MAX_POS - MIN_POS))
_M1 = float(1023.999 / (2.0 * MAX_SPEED))

_mesh = plsc.VectorSubcoreMesh(core_axis_name="c", subcore_axis_name="s")


@functools.partial(
    pl.kernel,
    mesh=_mesh,
    out_type=jax.ShapeDtypeStruct((N,), jnp.float32),
    scratch_types=[
        pltpu.VMEM((4, CHUNK), jnp.float32),
        pltpu.SemaphoreType.DMA,
        pltpu.SemaphoreType.DMA,
    ],
)
def _sc_lookup(inp_t_hbm, data_hbm, out_hbm, scr, sem_l, sem_g):
    wid = lax.axis_index("s") * NC + lax.axis_index("c")
    base = wid * CHUNK
    scr_i = scr.bitcast(jnp.int32)

    pltpu.async_copy(
        inp_t_hbm.at[:, pl.ds(base, CHUNK)], scr.at[pl.ds(0, 2), :], sem_l
    ).wait()

    b0 = jnp.float32(_B0)
    b1 = jnp.float32(_B1)
    m0 = jnp.float32(_M0)
    m1 = jnp.float32(_M1)

    flat = data_hbm.at[0, pl.ds(0, IDX_BLK)]

    g_cps = []
    for j in range(NBLK):
        blk = pl.ds(j * IDX_BLK, IDX_BLK)
        for g in range(j * GRP_PER_BLK, (j + 1) * GRP_PER_BLK):
            grp = pl.ds(g * LANES, LANES)
            pos = scr[0, grp]
            vel = scr[1, grp]
            r = ((pos + b0) * m0).astype(jnp.int32)
            c = ((vel + b1) * m1).astype(jnp.int32)
            scr_i[2, grp] = (
                ((r >> 3) << 13) + ((r & 7) << 7) + ((c >> 7) << 10) + (c & 127)
            )
        g_cps.append(
            pltpu.async_copy(flat.at[scr_i.at[2, blk]], scr.at[3, blk], sem_g)
        )
    for cp in g_cps:
        cp.wait()

    pltpu.sync_copy(scr.at[3], out_hbm.at[pl.ds(base, CHUNK)])


def kernel(inp, data):
    return _sc_lookup(inp.T, data)
